# baseline (speedup 1.0000x reference)
import functools

import jax
import jax.numpy as jnp
import numpy as np
from jax.experimental import pallas as pl
from jax.experimental.pallas import tpu as pltpu
from jax.experimental.pallas import tpu_sc as plsc

B, L, K = 2, 256, 30
NUM_RBF = 16
MAX_REL = 32
NA = 14
NP = NA * NA
NPP = 224
ROWS = 64
NE = ROWS * K

_f32 = jnp.float32
_i32 = jnp.int32


def _np_perm_p48():
    p = np.zeros((48, 128), np.float32)
    for a in range(NA):
        for c in range(3):
            p[a * 3 + c, c * 16 + a] = 1.0
    return p


def _np_expanders():
    ep = np.zeros((16, 256), np.float32)
    eq = np.zeros((16, 256), np.float32)
    for p in range(NA):
        for q in range(NA):
            ep[p, p * 16 + q] = 1.0
            eq[q, p * 16 + q] = 1.0
    return ep, eq


def _np_rep16():
    rp = np.zeros((16, NPP), np.float32)
    for q in range(NA):
        for r in range(NUM_RBF):
            rp[q, q * NUM_RBF + r] = 1.0
    return rp


_P48 = _np_perm_p48()
_EXP_P, _EXP_Q = _np_expanders()
_REP16 = _np_rep16()
_MU = np.linspace(0.0, 20.0, NUM_RBF).astype(np.float32)
_MU8 = (_MU.astype(np.float64) * 0.8).astype(np.float32)


def _stage1_body(xf_ref, cat_ref, sb_ref, wn_ref, bn_ref, gn_ref, betan_ref,
                 p48_ref, d_ref, x2cm_ref, v_ref):
    xf = xf_ref[0]
    n_at = xf[:, 0:3]
    ca = xf[:, 3:6]
    c_at = xf[:, 6:9]
    bb = ca - n_at
    cc = c_at - ca
    aa = jnp.concatenate([
        bb[:, 1:2] * cc[:, 2:3] - bb[:, 2:3] * cc[:, 1:2],
        bb[:, 2:3] * cc[:, 0:1] - bb[:, 0:1] * cc[:, 2:3],
        bb[:, 0:1] * cc[:, 1:2] - bb[:, 1:2] * cc[:, 0:1],
    ], axis=1)
    cb = -0.58273431 * aa + 0.56802827 * bb - 0.54067466 * cc + ca
    x2am = jnp.concatenate([xf[:, 0:12], cb, xf[:, 15:48]], axis=1)
    x2cm_ref[0] = jnp.dot(x2am, p48_ref[...], preferred_element_type=_f32)

    cat = cat_ref[0]
    d2 = None
    for c in range(3):
        row = jnp.broadcast_to(cat[c:c + 1, :], (L, L))
        col = jnp.broadcast_to(ca[:, c:c + 1], (L, L))
        s = (row - col) * (row - col)
        d2 = s if d2 is None else d2 + s
    d_ref[0] = jnp.sqrt(d2 + 1e-6)

    sb = sb_ref[0]
    s_col = sb[:, 0:1].astype(_i32)
    oh = (jax.lax.broadcasted_iota(_i32, (L, 21), 1) == s_col).astype(_f32)
    v_in = jnp.concatenate([oh, sb[:, 1:7], jnp.zeros((L, 5), _f32)], axis=1)
    v = jnp.dot(v_in, wn_ref[...], preferred_element_type=_f32) + bn_ref[0:1, :]
    mu = jnp.mean(v, axis=1, keepdims=True)
    var = jnp.mean((v - mu) ** 2, axis=1, keepdims=True)
    v_ref[0] = (v - mu) / jnp.sqrt(var + 1e-5) * gn_ref[0:1, :] + betan_ref[0:1, :]


_NC, _NS = 2, 16
_NW = _NC * _NS
_ROWS_PER_W = (B * L) // _NW


def _shuf(x, perm):
    dn = jax.lax.GatherDimensionNumbers(
        offset_dims=(), collapsed_slice_dims=(0,), start_index_map=(0,))
    return jax.lax.gather(
        x, perm, dn, (1,),
        mode=jax.lax.GatherScatterMode.PROMISE_IN_BOUNDS)


def _minbcast(x, perms):
    for p in perms:
        x = jnp.minimum(x, _shuf(x, p))
    return x


def _stage2_sc_body(d_hbm, out_hbm, vals_v, idx_v):
    wid = jax.lax.axis_index("s") * _NC + jax.lax.axis_index("c")
    iota = jax.lax.iota(_i32, 16)
    idxs = [iota + t * 16 for t in range(16)]
    perms = [jnp.reshape(iota ^ (1 << s), (16, 1)) for s in range(4)]

    def row_body(j, carry):
        row = wid * _ROWS_PER_W + j
        pltpu.sync_copy(d_hbm.at[pl.ds(row * L, L)], vals_v)
        chunks = tuple(vals_v[pl.ds(t * 16, 16)] for t in range(16))
        r0 = jnp.zeros((16,), _i32)

        def kstep(k, st):
            chs = list(st[0:16])
            r0, r1 = st[16], st[17]
            vs, ins = chs, list(idxs)
            while len(vs) > 1:
                nv, ni = [], []
                for t in range(0, len(vs), 2):
                    va, vb, ia, ib = vs[t], vs[t + 1], ins[t], ins[t + 1]
                    ta = (va < vb) | ((va == vb) & (ia < ib))
                    nv.append(jnp.where(ta, va, vb))
                    ni.append(jnp.where(ta, ia, ib))
                vs, ins = nv, ni
            m = _minbcast(vs[0], perms)
            mi = _minbcast(jnp.where(vs[0] == m, ins[0], jnp.int32(1 << 20)),
                           perms)
            r0 = jnp.where(iota == k, mi, r0)
            r1 = jnp.where(iota == k - 16, mi, r1)
            chs = [jnp.where(idxs[t] == mi, jnp.float32(jnp.inf), chs[t])
                   for t in range(16)]
            return (*chs, r0, r1)

        st = jax.lax.fori_loop(0, K, kstep, (*chunks, r0, r0))
        idx_v[pl.ds(0, 16)] = st[16]
        idx_v[pl.ds(16, 16)] = st[17]
        pltpu.sync_copy(idx_v, out_hbm.at[pl.ds(row * 32, 32)])
        return carry

    jax.lax.fori_loop(0, _ROWS_PER_W, row_body, 0)


def _stage3_body(eidx_ref, x2cm_ref, x2blk_ref, wpos_ref, wrbf_ref, expp_ref,
                 expq_ref, rep_ref, mu_ref, be_ref, ge_ref, betae_ref, e_ref):
    ecol = eidx_ref[0, 0][:, 0:1]
    base = pl.program_id(1) * ROWS
    erow = jax.lax.broadcasted_iota(_i32, (NE, 1), 0)
    rr = erow // K
    self_col = base + rr
    iota_l = jax.lax.broadcasted_iota(_i32, (NE, L), 1)
    oh_nb = (iota_l == ecol).astype(_f32)
    oh_self = (jax.lax.broadcasted_iota(_i32, (NE, ROWS), 1) == rr).astype(_f32)
    x2 = x2cm_ref[0]
    xnb = jnp.dot(oh_nb, x2, preferred_element_type=_f32)
    xi = jnp.dot(oh_self, x2blk_ref[0], preferred_element_type=_f32)

    relpos = jnp.clip(ecol - self_col + MAX_REL, 0, 2 * MAX_REL)
    oh_pos = (jax.lax.broadcasted_iota(_i32, (NE, 128), 1) == relpos).astype(_f32)
    acc = jnp.dot(oh_pos, wpos_ref[...], preferred_element_type=_f32) + be_ref[0:1, :]

    d2 = None
    for c in range(3):
        a = jnp.dot(xi[:, c * 16:(c + 1) * 16], expp_ref[...],
                    preferred_element_type=_f32)
        b = jnp.dot(xnb[:, c * 16:(c + 1) * 16], expq_ref[...],
                    preferred_element_type=_f32)
        s = (a - b) * (a - b)
        d2 = s if d2 is None else d2 + s
    dps = jnp.sqrt(d2 + 1e-6) * jnp.float32(0.8)

    mu8 = mu_ref[0:1, :]
    for p in range(NA):
        dpe = jnp.dot(dps[:, p * 16:(p + 1) * 16], rep_ref[...],
                      preferred_element_type=_f32)
        z = dpe - mu8
        rbf = jnp.exp(-(z * z))
        acc = acc + jnp.dot(rbf, wrbf_ref[p], preferred_element_type=_f32)

    m = jnp.mean(acc, axis=1, keepdims=True)
    var = jnp.mean((acc - m) ** 2, axis=1, keepdims=True)
    e_ref[0, 0] = (acc - m) / jnp.sqrt(var + 1e-5) * ge_ref[0:1, :] + betae_ref[0:1, :]


def kernel(X, S, BB_D, mask, W_node, b_node, g_node, beta_node,
           W_edge, b_edge, g_edge, beta_edge):
    del mask
    xf = jnp.pad(X.reshape(B, L, 42), ((0, 0), (0, 0), (0, 6)))
    cat = jnp.pad(jnp.transpose(X[:, :, 1, :], (0, 2, 1)), ((0, 0), (0, 5), (0, 0)))
    sb = jnp.concatenate([S.astype(_f32)[..., None], BB_D.reshape(B, L, 6),
                          jnp.zeros((B, L, 1), _f32)], axis=-1)
    wn = jnp.pad(W_node, ((0, 5), (0, 0)))
    row8 = lambda v: jnp.broadcast_to(v[None, :], (8, v.shape[0]))
    p48 = jnp.asarray(_P48)

    d_mat, x2cm, v_out = pl.pallas_call(
        _stage1_body,
        grid=(B,),
        in_specs=[
            pl.BlockSpec((1, L, 48), lambda b: (b, 0, 0)),
            pl.BlockSpec((1, 8, L), lambda b: (b, 0, 0)),
            pl.BlockSpec((1, L, 8), lambda b: (b, 0, 0)),
            pl.BlockSpec((32, 128), lambda b: (0, 0)),
            pl.BlockSpec((8, 128), lambda b: (0, 0)),
            pl.BlockSpec((8, 128), lambda b: (0, 0)),
            pl.BlockSpec((8, 128), lambda b: (0, 0)),
            pl.BlockSpec((48, 128), lambda b: (0, 0)),
        ],
        out_specs=[
            pl.BlockSpec((1, L, L), lambda b: (b, 0, 0)),
            pl.BlockSpec((1, L, 128), lambda b: (b, 0, 0)),
            pl.BlockSpec((1, L, 128), lambda b: (b, 0, 0)),
        ],
        out_shape=[
            jax.ShapeDtypeStruct((B, L, L), _f32),
            jax.ShapeDtypeStruct((B, L, 128), _f32),
            jax.ShapeDtypeStruct((B, L, 128), _f32),
        ],
    )(xf, cat, sb, wn, row8(b_node), row8(g_node), row8(beta_node), p48)

    sc_mesh = plsc.VectorSubcoreMesh(core_axis_name="c", subcore_axis_name="s")
    eidx_flat = pl.kernel(
        _stage2_sc_body,
        out_type=jax.ShapeDtypeStruct((B * L * 32,), _i32),
        mesh=sc_mesh,
        scratch_types=[
            pltpu.VMEM((L,), _f32),
            pltpu.VMEM((32,), _i32),
        ],
    )(d_mat.reshape(B * L * L))

    e_idx = eidx_flat.reshape(B, L, 32)[:, :, :K]
    nblk = L // ROWS
    eidx4 = e_idx.reshape(B, nblk, NE, 1)

    w_pos = jnp.pad(W_edge[:65], ((0, 63), (0, 0)))
    w_rbf = W_edge[65:].reshape(NA, NPP, 128)
    mu_t = row8(jnp.asarray(np.tile(_MU8, NA)))

    e_blocks = pl.pallas_call(
        _stage3_body,
        grid=(B, nblk),
        in_specs=[
            pl.BlockSpec((1, 1, NE, 1), lambda b, i: (b, i, 0, 0)),
            pl.BlockSpec((1, L, 128), lambda b, i: (b, 0, 0)),
            pl.BlockSpec((1, ROWS, 128), lambda b, i: (b, i, 0)),
            pl.BlockSpec((128, 128), lambda b, i: (0, 0)),
            pl.BlockSpec((NA, NPP, 128), lambda b, i: (0, 0, 0)),
            pl.BlockSpec((16, 256), lambda b, i: (0, 0)),
            pl.BlockSpec((16, 256), lambda b, i: (0, 0)),
            pl.BlockSpec((16, NPP), lambda b, i: (0, 0)),
            pl.BlockSpec((8, NPP), lambda b, i: (0, 0)),
            pl.BlockSpec((8, 128), lambda b, i: (0, 0)),
            pl.BlockSpec((8, 128), lambda b, i: (0, 0)),
            pl.BlockSpec((8, 128), lambda b, i: (0, 0)),
        ],
        out_specs=pl.BlockSpec((1, 1, NE, 128), lambda b, i: (b, i, 0, 0)),
        out_shape=jax.ShapeDtypeStruct((B, nblk, NE, 128), _f32),
    )(eidx4, x2cm, x2cm, w_pos, w_rbf, jnp.asarray(_EXP_P), jnp.asarray(_EXP_Q),
      jnp.asarray(_REP16), mu_t, row8(b_edge), row8(g_edge), row8(beta_edge))

    e_out = e_blocks.reshape(B, nblk, ROWS, K, 128).reshape(B, L, K, 128)
    return v_out, e_out, e_idx, X


# speedup vs baseline: 1.0626x; 1.0626x over previous
import functools

import jax
import jax.numpy as jnp
import numpy as np
from jax.experimental import pallas as pl
from jax.experimental.pallas import tpu as pltpu
from jax.experimental.pallas import tpu_sc as plsc

B, L, K = 2, 256, 30
NUM_RBF = 16
MAX_REL = 32
NA = 14
NP = NA * NA
NPP = 224
ROWS = 32
NE = ROWS * K

_f32 = jnp.float32
_i32 = jnp.int32


def _np_perm_p48():
    p = np.zeros((48, 128), np.float32)
    for a in range(NA):
        for c in range(3):
            p[a * 3 + c, c * 16 + a] = 1.0
    return p


def _np_expanders():
    ep = np.zeros((16, 256), np.float32)
    eq = np.zeros((16, 256), np.float32)
    for p in range(NA):
        for q in range(NA):
            ep[p, p * 16 + q] = 1.0
            eq[q, p * 16 + q] = 1.0
    return ep, eq


def _np_rep16():
    rp = np.zeros((16, NPP), np.float32)
    for q in range(NA):
        for r in range(NUM_RBF):
            rp[q, q * NUM_RBF + r] = 1.0
    return rp


_P48 = _np_perm_p48()
_EXP_P, _EXP_Q = _np_expanders()
_REP16 = _np_rep16()
_MU = np.linspace(0.0, 20.0, NUM_RBF).astype(np.float32)
_MU8 = (_MU.astype(np.float64) * 0.8).astype(np.float32)


def _stage1_body(xf_ref, cat_ref, sb_ref, wn_ref, bn_ref, gn_ref, betan_ref,
                 p48_ref, d_ref, x2cm_ref, v_ref):
    xf = xf_ref[0]
    n_at = xf[:, 0:3]
    ca = xf[:, 3:6]
    c_at = xf[:, 6:9]
    bb = ca - n_at
    cc = c_at - ca
    aa = jnp.concatenate([
        bb[:, 1:2] * cc[:, 2:3] - bb[:, 2:3] * cc[:, 1:2],
        bb[:, 2:3] * cc[:, 0:1] - bb[:, 0:1] * cc[:, 2:3],
        bb[:, 0:1] * cc[:, 1:2] - bb[:, 1:2] * cc[:, 0:1],
    ], axis=1)
    cb = -0.58273431 * aa + 0.56802827 * bb - 0.54067466 * cc + ca
    x2am = jnp.concatenate([xf[:, 0:12], cb, xf[:, 15:48]], axis=1)
    x2cm_ref[0] = jnp.dot(x2am, p48_ref[...], preferred_element_type=_f32)

    cat = cat_ref[0]
    d2 = None
    for c in range(3):
        row = jnp.broadcast_to(cat[c:c + 1, :], (L, L))
        col = jnp.broadcast_to(ca[:, c:c + 1], (L, L))
        s = (row - col) * (row - col)
        d2 = s if d2 is None else d2 + s
    d_ref[0] = jnp.sqrt(d2 + 1e-6)

    sb = sb_ref[0]
    s_col = sb[:, 0:1].astype(_i32)
    oh = (jax.lax.broadcasted_iota(_i32, (L, 21), 1) == s_col).astype(_f32)
    v_in = jnp.concatenate([oh, sb[:, 1:7], jnp.zeros((L, 5), _f32)], axis=1)
    v = jnp.dot(v_in, wn_ref[...], preferred_element_type=_f32) + bn_ref[0:1, :]
    mu = jnp.mean(v, axis=1, keepdims=True)
    var = jnp.mean((v - mu) ** 2, axis=1, keepdims=True)
    v_ref[0] = (v - mu) / jnp.sqrt(var + 1e-5) * gn_ref[0:1, :] + betan_ref[0:1, :]


_NC, _NS = 2, 16
_NW = _NC * _NS
_ROWS_PER_W = (B * L) // _NW


def _shuf(x, perm):
    dn = jax.lax.GatherDimensionNumbers(
        offset_dims=(), collapsed_slice_dims=(0,), start_index_map=(0,))
    return jax.lax.gather(
        x, perm, dn, (1,),
        mode=jax.lax.GatherScatterMode.PROMISE_IN_BOUNDS)


def _minbcast(x, perms):
    for p in perms:
        x = jnp.minimum(x, _shuf(x, p))
    return x


def _stage2_sc_body(d_hbm, out_hbm, vals_v, idx_v):
    wid = jax.lax.axis_index("s") * _NC + jax.lax.axis_index("c")
    iota = jax.lax.iota(_i32, 16)
    idxs = [iota + t * 16 for t in range(16)]
    perms = [jnp.reshape(iota ^ (1 << s), (16, 1)) for s in range(4)]

    def row_body(j, carry):
        row = wid * _ROWS_PER_W + j
        pltpu.sync_copy(d_hbm.at[pl.ds(row * L, L)], vals_v)
        chunks = tuple(vals_v[pl.ds(t * 16, 16)] for t in range(16))
        r0 = jnp.zeros((16,), _i32)

        def kstep(k, st):
            chs = list(st[0:16])
            r0, r1 = st[16], st[17]
            vs, ins = chs, list(idxs)
            while len(vs) > 1:
                nv, ni = [], []
                for t in range(0, len(vs), 2):
                    va, vb, ia, ib = vs[t], vs[t + 1], ins[t], ins[t + 1]
                    ta = va <= vb
                    nv.append(jnp.where(ta, va, vb))
                    ni.append(jnp.where(ta, ia, ib))
                vs, ins = nv, ni
            m = _minbcast(vs[0], perms)
            mi = _minbcast(jnp.where(vs[0] == m, ins[0], jnp.int32(1 << 20)),
                           perms)
            r0 = jnp.where(iota == k, mi, r0)
            r1 = jnp.where(iota == k - 16, mi, r1)
            chs = [jnp.where(idxs[t] == mi, jnp.float32(jnp.inf), chs[t])
                   for t in range(16)]
            return (*chs, r0, r1)

        st = jax.lax.fori_loop(0, K, kstep, (*chunks, r0, r0))
        idx_v[pl.ds(0, 16)] = st[16]
        idx_v[pl.ds(16, 16)] = st[17]
        pltpu.sync_copy(idx_v, out_hbm.at[pl.ds(row * 32, 32)])
        return carry

    jax.lax.fori_loop(0, _ROWS_PER_W, row_body, 0)


def _stage3_body(eidx_ref, x2cm_ref, x2blk_ref, wpos_ref, wrbf_ref, expp_ref,
                 expq_ref, rep_ref, mu_ref, be_ref, ge_ref, betae_ref, e_ref):
    ecol = eidx_ref[0, 0][:, 0:1]
    base = pl.program_id(1) * ROWS
    erow = jax.lax.broadcasted_iota(_i32, (NE, 1), 0)
    rr = erow // K
    self_col = base + rr
    iota_l = jax.lax.broadcasted_iota(_i32, (NE, L), 1)
    oh_nb = (iota_l == ecol).astype(_f32)
    oh_self = (jax.lax.broadcasted_iota(_i32, (NE, ROWS), 1) == rr).astype(_f32)
    x2 = x2cm_ref[0]
    xnb = jnp.dot(oh_nb, x2, preferred_element_type=_f32)
    xi = jnp.dot(oh_self, x2blk_ref[0], preferred_element_type=_f32)

    relpos = jnp.clip(ecol - self_col + MAX_REL, 0, 2 * MAX_REL)
    oh_pos = (jax.lax.broadcasted_iota(_i32, (NE, 128), 1) == relpos).astype(_f32)
    acc = jnp.dot(oh_pos, wpos_ref[...], preferred_element_type=_f32) + be_ref[0:1, :]

    d2 = None
    for c in range(3):
        a = jnp.dot(xi[:, c * 16:(c + 1) * 16], expp_ref[...],
                    preferred_element_type=_f32)
        b = jnp.dot(xnb[:, c * 16:(c + 1) * 16], expq_ref[...],
                    preferred_element_type=_f32)
        s = (a - b) * (a - b)
        d2 = s if d2 is None else d2 + s
    dps = jnp.sqrt(d2 + 1e-6) * jnp.float32(0.8)

    mu8 = mu_ref[0:1, :]
    for p in range(NA):
        dpe = jnp.dot(dps[:, p * 16:(p + 1) * 16], rep_ref[...],
                      preferred_element_type=_f32)
        z = dpe - mu8
        rbf = jnp.exp(-(z * z))
        acc = acc + jnp.dot(rbf, wrbf_ref[p], preferred_element_type=_f32)

    m = jnp.mean(acc, axis=1, keepdims=True)
    var = jnp.mean((acc - m) ** 2, axis=1, keepdims=True)
    e_ref[0, 0] = (acc - m) / jnp.sqrt(var + 1e-5) * ge_ref[0:1, :] + betae_ref[0:1, :]


def kernel(X, S, BB_D, mask, W_node, b_node, g_node, beta_node,
           W_edge, b_edge, g_edge, beta_edge):
    del mask
    xf = jnp.pad(X.reshape(B, L, 42), ((0, 0), (0, 0), (0, 6)))
    cat = jnp.pad(jnp.transpose(X[:, :, 1, :], (0, 2, 1)), ((0, 0), (0, 5), (0, 0)))
    sb = jnp.concatenate([S.astype(_f32)[..., None], BB_D.reshape(B, L, 6),
                          jnp.zeros((B, L, 1), _f32)], axis=-1)
    wn = jnp.pad(W_node, ((0, 5), (0, 0)))
    row8 = lambda v: jnp.broadcast_to(v[None, :], (8, v.shape[0]))
    p48 = jnp.asarray(_P48)

    d_mat, x2cm, v_out = pl.pallas_call(
        _stage1_body,
        grid=(B,),
        in_specs=[
            pl.BlockSpec((1, L, 48), lambda b: (b, 0, 0)),
            pl.BlockSpec((1, 8, L), lambda b: (b, 0, 0)),
            pl.BlockSpec((1, L, 8), lambda b: (b, 0, 0)),
            pl.BlockSpec((32, 128), lambda b: (0, 0)),
            pl.BlockSpec((8, 128), lambda b: (0, 0)),
            pl.BlockSpec((8, 128), lambda b: (0, 0)),
            pl.BlockSpec((8, 128), lambda b: (0, 0)),
            pl.BlockSpec((48, 128), lambda b: (0, 0)),
        ],
        out_specs=[
            pl.BlockSpec((1, L, L), lambda b: (b, 0, 0)),
            pl.BlockSpec((1, L, 128), lambda b: (b, 0, 0)),
            pl.BlockSpec((1, L, 128), lambda b: (b, 0, 0)),
        ],
        out_shape=[
            jax.ShapeDtypeStruct((B, L, L), _f32),
            jax.ShapeDtypeStruct((B, L, 128), _f32),
            jax.ShapeDtypeStruct((B, L, 128), _f32),
        ],
    )(xf, cat, sb, wn, row8(b_node), row8(g_node), row8(beta_node), p48)

    sc_mesh = plsc.VectorSubcoreMesh(core_axis_name="c", subcore_axis_name="s")
    eidx_flat = pl.kernel(
        _stage2_sc_body,
        out_type=jax.ShapeDtypeStruct((B * L * 32,), _i32),
        mesh=sc_mesh,
        scratch_types=[
            pltpu.VMEM((L,), _f32),
            pltpu.VMEM((32,), _i32),
        ],
    )(d_mat.reshape(B * L * L))

    e_idx = eidx_flat.reshape(B, L, 32)[:, :, :K]
    nblk = L // ROWS
    eidx4 = e_idx.reshape(B, nblk, NE, 1)

    w_pos = jnp.pad(W_edge[:65], ((0, 63), (0, 0)))
    w_rbf = W_edge[65:].reshape(NA, NPP, 128)
    mu_t = row8(jnp.asarray(np.tile(_MU8, NA)))

    e_blocks = pl.pallas_call(
        _stage3_body,
        grid=(B, nblk),
        in_specs=[
            pl.BlockSpec((1, 1, NE, 1), lambda b, i: (b, i, 0, 0)),
            pl.BlockSpec((1, L, 128), lambda b, i: (b, 0, 0)),
            pl.BlockSpec((1, ROWS, 128), lambda b, i: (b, i, 0)),
            pl.BlockSpec((128, 128), lambda b, i: (0, 0)),
            pl.BlockSpec((NA, NPP, 128), lambda b, i: (0, 0, 0)),
            pl.BlockSpec((16, 256), lambda b, i: (0, 0)),
            pl.BlockSpec((16, 256), lambda b, i: (0, 0)),
            pl.BlockSpec((16, NPP), lambda b, i: (0, 0)),
            pl.BlockSpec((8, NPP), lambda b, i: (0, 0)),
            pl.BlockSpec((8, 128), lambda b, i: (0, 0)),
            pl.BlockSpec((8, 128), lambda b, i: (0, 0)),
            pl.BlockSpec((8, 128), lambda b, i: (0, 0)),
        ],
        out_specs=pl.BlockSpec((1, 1, NE, 128), lambda b, i: (b, i, 0, 0)),
        out_shape=jax.ShapeDtypeStruct((B, nblk, NE, 128), _f32),
    )(eidx4, x2cm, x2cm, w_pos, w_rbf, jnp.asarray(_EXP_P), jnp.asarray(_EXP_Q),
      jnp.asarray(_REP16), mu_t, row8(b_edge), row8(g_edge), row8(beta_edge))

    e_out = e_blocks.reshape(B, nblk, ROWS, K, 128).reshape(B, L, K, 128)
    return v_out, e_out, e_idx, X


# speedup vs baseline: 1.0683x; 1.0053x over previous
import functools

import jax
import jax.numpy as jnp
import numpy as np
from jax.experimental import pallas as pl
from jax.experimental.pallas import tpu as pltpu
from jax.experimental.pallas import tpu_sc as plsc

B, L, K = 2, 256, 30
NUM_RBF = 16
MAX_REL = 32
NA = 14
NP = NA * NA
NPP = 224
ROWS = 32
NE = ROWS * K

_f32 = jnp.float32
_i32 = jnp.int32


def _np_perm_p48():
    p = np.zeros((48, 128), np.float32)
    for a in range(NA):
        for c in range(3):
            p[a * 3 + c, c * 16 + a] = 1.0
    return p


def _np_expanders():
    ep = np.zeros((16, 256), np.float32)
    eq = np.zeros((16, 256), np.float32)
    for p in range(NA):
        for q in range(NA):
            ep[p, p * 16 + q] = 1.0
            eq[q, p * 16 + q] = 1.0
    return ep, eq


def _np_rep16():
    rp = np.zeros((16, NPP), np.float32)
    for q in range(NA):
        for r in range(NUM_RBF):
            rp[q, q * NUM_RBF + r] = 1.0
    return rp


_P48 = _np_perm_p48()
_EXP_P, _EXP_Q = _np_expanders()
_REP16 = _np_rep16()
_MU = np.linspace(0.0, 20.0, NUM_RBF).astype(np.float32)
_MU8 = (_MU.astype(np.float64) * 0.8).astype(np.float32)


def _stage1_body(xf_ref, cat_ref, sb_ref, wn_ref, bn_ref, gn_ref, betan_ref,
                 p48_ref, d_ref, x2cm_ref, v_ref):
    xf = xf_ref[0]
    n_at = xf[:, 0:3]
    ca = xf[:, 3:6]
    c_at = xf[:, 6:9]
    bb = ca - n_at
    cc = c_at - ca
    aa = jnp.concatenate([
        bb[:, 1:2] * cc[:, 2:3] - bb[:, 2:3] * cc[:, 1:2],
        bb[:, 2:3] * cc[:, 0:1] - bb[:, 0:1] * cc[:, 2:3],
        bb[:, 0:1] * cc[:, 1:2] - bb[:, 1:2] * cc[:, 0:1],
    ], axis=1)
    cb = -0.58273431 * aa + 0.56802827 * bb - 0.54067466 * cc + ca
    x2am = jnp.concatenate([xf[:, 0:12], cb, xf[:, 15:48]], axis=1)
    x2cm_ref[0] = jnp.dot(x2am, p48_ref[...], preferred_element_type=_f32)

    cat = cat_ref[0]
    d2 = None
    for c in range(3):
        row = jnp.broadcast_to(cat[c:c + 1, :], (L, L))
        col = jnp.broadcast_to(ca[:, c:c + 1], (L, L))
        s = (row - col) * (row - col)
        d2 = s if d2 is None else d2 + s
    d_ref[0] = jnp.sqrt(d2 + 1e-6)

    sb = sb_ref[0]
    s_col = sb[:, 0:1].astype(_i32)
    oh = (jax.lax.broadcasted_iota(_i32, (L, 21), 1) == s_col).astype(_f32)
    v_in = jnp.concatenate([oh, sb[:, 1:7], jnp.zeros((L, 5), _f32)], axis=1)
    v = jnp.dot(v_in, wn_ref[...], preferred_element_type=_f32) + bn_ref[0:1, :]
    mu = jnp.mean(v, axis=1, keepdims=True)
    var = jnp.mean((v - mu) ** 2, axis=1, keepdims=True)
    v_ref[0] = (v - mu) / jnp.sqrt(var + 1e-5) * gn_ref[0:1, :] + betan_ref[0:1, :]


_NC, _NS = 2, 16
_NW = _NC * _NS
_ROWS_PER_W = (B * L) // _NW


def _shuf(x, perm):
    dn = jax.lax.GatherDimensionNumbers(
        offset_dims=(), collapsed_slice_dims=(0,), start_index_map=(0,))
    return jax.lax.gather(
        x, perm, dn, (1,),
        mode=jax.lax.GatherScatterMode.PROMISE_IN_BOUNDS)


def _minbcast(x, perms):
    for p in perms:
        x = jnp.minimum(x, _shuf(x, p))
    return x


def _stage2_sc_body(d_hbm, out_hbm, vals_v, vals_v2, idx_v, idx_v2):
    wid = jax.lax.axis_index("s") * _NC + jax.lax.axis_index("c")
    iota = jax.lax.iota(_i32, 16)
    idxs = [iota + t * 16 for t in range(16)]
    perms = [jnp.reshape(iota ^ (1 << s), (16, 1)) for s in range(4)]

    half = _ROWS_PER_W // 2

    def extract(chs, ins, k, r0, r1):
        vs = list(chs)
        ins = list(ins)
        while len(vs) > 1:
            nv, ni = [], []
            for t in range(0, len(vs), 2):
                va, vb, ia, ib = vs[t], vs[t + 1], ins[t], ins[t + 1]
                ta = va <= vb
                nv.append(jnp.where(ta, va, vb))
                ni.append(jnp.where(ta, ia, ib))
            vs, ins = nv, ni
        m = _minbcast(vs[0], perms)
        mi = _minbcast(jnp.where(vs[0] == m, ins[0], jnp.int32(1 << 20)),
                       perms)
        r0 = jnp.where(iota == k, mi, r0)
        r1 = jnp.where(iota == k - 16, mi, r1)
        chs = [jnp.where(idxs[t] == mi, jnp.float32(jnp.inf), chs[t])
               for t in range(16)]
        return chs, r0, r1

    def row_body(j, carry):
        row_a = wid * _ROWS_PER_W + j
        row_b = row_a + half
        pltpu.sync_copy(d_hbm.at[pl.ds(row_a * L, L)], vals_v)
        pltpu.sync_copy(d_hbm.at[pl.ds(row_b * L, L)], vals_v2)
        chunks_a = tuple(vals_v[pl.ds(t * 16, 16)] for t in range(16))
        chunks_b = tuple(vals_v2[pl.ds(t * 16, 16)] for t in range(16))
        z16 = jnp.zeros((16,), _i32)

        def kstep(k, st):
            ca, cb = list(st[0:16]), list(st[16:32])
            a0, a1, b0, b1 = st[32], st[33], st[34], st[35]
            ca, a0, a1 = extract(ca, idxs, k, a0, a1)
            cb, b0, b1 = extract(cb, idxs, k, b0, b1)
            return (*ca, *cb, a0, a1, b0, b1)

        st = jax.lax.fori_loop(0, K, kstep,
                               (*chunks_a, *chunks_b, z16, z16, z16, z16))
        idx_v[pl.ds(0, 16)] = st[32]
        idx_v[pl.ds(16, 16)] = st[33]
        pltpu.sync_copy(idx_v, out_hbm.at[pl.ds(row_a * 32, 32)])
        idx_v2[pl.ds(0, 16)] = st[34]
        idx_v2[pl.ds(16, 16)] = st[35]
        pltpu.sync_copy(idx_v2, out_hbm.at[pl.ds(row_b * 32, 32)])
        return carry

    jax.lax.fori_loop(0, half, row_body, 0)


def _stage3_body(eidx_ref, x2cm_ref, x2blk_ref, wpos_ref, wrbf_ref, expp_ref,
                 expq_ref, rep_ref, mu_ref, be_ref, ge_ref, betae_ref, e_ref):
    ecol = eidx_ref[0, 0][:, 0:1]
    base = pl.program_id(1) * ROWS
    erow = jax.lax.broadcasted_iota(_i32, (NE, 1), 0)
    rr = erow // K
    self_col = base + rr
    iota_l = jax.lax.broadcasted_iota(_i32, (NE, L), 1)
    oh_nb = (iota_l == ecol).astype(_f32)
    oh_self = (jax.lax.broadcasted_iota(_i32, (NE, ROWS), 1) == rr).astype(_f32)
    x2 = x2cm_ref[0]
    xnb = jnp.dot(oh_nb, x2, preferred_element_type=_f32)
    xi = jnp.dot(oh_self, x2blk_ref[0], preferred_element_type=_f32)

    relpos = jnp.clip(ecol - self_col + MAX_REL, 0, 2 * MAX_REL)
    oh_pos = (jax.lax.broadcasted_iota(_i32, (NE, 128), 1) == relpos).astype(_f32)
    acc = jnp.dot(oh_pos, wpos_ref[...], preferred_element_type=_f32) + be_ref[0:1, :]

    d2 = None
    for c in range(3):
        a = jnp.dot(xi[:, c * 16:(c + 1) * 16], expp_ref[...],
                    preferred_element_type=_f32)
        b = jnp.dot(xnb[:, c * 16:(c + 1) * 16], expq_ref[...],
                    preferred_element_type=_f32)
        s = (a - b) * (a - b)
        d2 = s if d2 is None else d2 + s
    dps = jnp.sqrt(d2 + 1e-6) * jnp.float32(0.8)

    mu8 = mu_ref[0:1, :]
    for p in range(NA):
        dpe = jnp.dot(dps[:, p * 16:(p + 1) * 16], rep_ref[...],
                      preferred_element_type=_f32)
        z = dpe - mu8
        rbf = jnp.exp(-(z * z))
        acc = acc + jnp.dot(rbf, wrbf_ref[p], preferred_element_type=_f32)

    m = jnp.mean(acc, axis=1, keepdims=True)
    var = jnp.mean((acc - m) ** 2, axis=1, keepdims=True)
    e_ref[0, 0] = (acc - m) / jnp.sqrt(var + 1e-5) * ge_ref[0:1, :] + betae_ref[0:1, :]


def kernel(X, S, BB_D, mask, W_node, b_node, g_node, beta_node,
           W_edge, b_edge, g_edge, beta_edge):
    del mask
    xf = jnp.pad(X.reshape(B, L, 42), ((0, 0), (0, 0), (0, 6)))
    cat = jnp.pad(jnp.transpose(X[:, :, 1, :], (0, 2, 1)), ((0, 0), (0, 5), (0, 0)))
    sb = jnp.concatenate([S.astype(_f32)[..., None], BB_D.reshape(B, L, 6),
                          jnp.zeros((B, L, 1), _f32)], axis=-1)
    wn = jnp.pad(W_node, ((0, 5), (0, 0)))
    row8 = lambda v: jnp.broadcast_to(v[None, :], (8, v.shape[0]))
    p48 = jnp.asarray(_P48)

    d_mat, x2cm, v_out = pl.pallas_call(
        _stage1_body,
        grid=(B,),
        in_specs=[
            pl.BlockSpec((1, L, 48), lambda b: (b, 0, 0)),
            pl.BlockSpec((1, 8, L), lambda b: (b, 0, 0)),
            pl.BlockSpec((1, L, 8), lambda b: (b, 0, 0)),
            pl.BlockSpec((32, 128), lambda b: (0, 0)),
            pl.BlockSpec((8, 128), lambda b: (0, 0)),
            pl.BlockSpec((8, 128), lambda b: (0, 0)),
            pl.BlockSpec((8, 128), lambda b: (0, 0)),
            pl.BlockSpec((48, 128), lambda b: (0, 0)),
        ],
        out_specs=[
            pl.BlockSpec((1, L, L), lambda b: (b, 0, 0)),
            pl.BlockSpec((1, L, 128), lambda b: (b, 0, 0)),
            pl.BlockSpec((1, L, 128), lambda b: (b, 0, 0)),
        ],
        out_shape=[
            jax.ShapeDtypeStruct((B, L, L), _f32),
            jax.ShapeDtypeStruct((B, L, 128), _f32),
            jax.ShapeDtypeStruct((B, L, 128), _f32),
        ],
    )(xf, cat, sb, wn, row8(b_node), row8(g_node), row8(beta_node), p48)

    sc_mesh = plsc.VectorSubcoreMesh(core_axis_name="c", subcore_axis_name="s")
    eidx_flat = pl.kernel(
        _stage2_sc_body,
        out_type=jax.ShapeDtypeStruct((B * L * 32,), _i32),
        mesh=sc_mesh,
        scratch_types=[
            pltpu.VMEM((L,), _f32),
            pltpu.VMEM((L,), _f32),
            pltpu.VMEM((32,), _i32),
            pltpu.VMEM((32,), _i32),
        ],
    )(d_mat.reshape(B * L * L))

    e_idx = eidx_flat.reshape(B, L, 32)[:, :, :K]
    nblk = L // ROWS
    eidx4 = e_idx.reshape(B, nblk, NE, 1)

    w_pos = jnp.pad(W_edge[:65], ((0, 63), (0, 0)))
    w_rbf = W_edge[65:].reshape(NA, NPP, 128)
    mu_t = row8(jnp.asarray(np.tile(_MU8, NA)))

    e_blocks = pl.pallas_call(
        _stage3_body,
        grid=(B, nblk),
        in_specs=[
            pl.BlockSpec((1, 1, NE, 1), lambda b, i: (b, i, 0, 0)),
            pl.BlockSpec((1, L, 128), lambda b, i: (b, 0, 0)),
            pl.BlockSpec((1, ROWS, 128), lambda b, i: (b, i, 0)),
            pl.BlockSpec((128, 128), lambda b, i: (0, 0)),
            pl.BlockSpec((NA, NPP, 128), lambda b, i: (0, 0, 0)),
            pl.BlockSpec((16, 256), lambda b, i: (0, 0)),
            pl.BlockSpec((16, 256), lambda b, i: (0, 0)),
            pl.BlockSpec((16, NPP), lambda b, i: (0, 0)),
            pl.BlockSpec((8, NPP), lambda b, i: (0, 0)),
            pl.BlockSpec((8, 128), lambda b, i: (0, 0)),
            pl.BlockSpec((8, 128), lambda b, i: (0, 0)),
            pl.BlockSpec((8, 128), lambda b, i: (0, 0)),
        ],
        out_specs=pl.BlockSpec((1, 1, NE, 128), lambda b, i: (b, i, 0, 0)),
        out_shape=jax.ShapeDtypeStruct((B, nblk, NE, 128), _f32),
    )(eidx4, x2cm, x2cm, w_pos, w_rbf, jnp.asarray(_EXP_P), jnp.asarray(_EXP_Q),
      jnp.asarray(_REP16), mu_t, row8(b_edge), row8(g_edge), row8(beta_edge))

    e_out = e_blocks.reshape(B, nblk, ROWS, K, 128).reshape(B, L, K, 128)
    return v_out, e_out, e_idx, X


# speedup vs baseline: 1.0689x; 1.0006x over previous
import jax
import jax.numpy as jnp
import numpy as np
from jax.experimental import pallas as pl
from jax.experimental.pallas import tpu as pltpu
from jax.experimental.pallas import tpu_sc as plsc

B, L, K = 2, 256, 30
NUM_RBF = 16
MAX_REL = 32
NA = 14
NP = NA * NA
NPP = 224
ROWS = 32
NE = ROWS * K

_f32 = jnp.float32
_i32 = jnp.int32


def _np_perm_p48():
    p = np.zeros((48, 128), np.float32)
    for a in range(NA):
        for c in range(3):
            p[a * 3 + c, c * 16 + a] = 1.0
    return p


def _np_expanders():
    ep = np.zeros((16, 256), np.float32)
    eq = np.zeros((16, 256), np.float32)
    for p in range(NA):
        for q in range(NA):
            ep[p, p * 16 + q] = 1.0
            eq[q, p * 16 + q] = 1.0
    return ep, eq


def _np_rep16():
    rp = np.zeros((16, NPP), np.float32)
    for q in range(NA):
        for r in range(NUM_RBF):
            rp[q, q * NUM_RBF + r] = 1.0
    return rp


_P48 = _np_perm_p48()
_EXP_P, _EXP_Q = _np_expanders()
_REP16 = _np_rep16()
_MU = np.linspace(0.0, 20.0, NUM_RBF).astype(np.float32)
_MU8 = (_MU.astype(np.float64) * 0.8).astype(np.float32)


def _stage1_body(xf_ref, cat_ref, sb_ref, wn_ref, bn_ref, gn_ref, betan_ref,
                 p48_ref, d_ref, x2cm_ref, v_ref):
    xf = xf_ref[0]
    n_at = xf[:, 0:3]
    ca = xf[:, 3:6]
    c_at = xf[:, 6:9]
    bb = ca - n_at
    cc = c_at - ca
    aa = jnp.concatenate([
        bb[:, 1:2] * cc[:, 2:3] - bb[:, 2:3] * cc[:, 1:2],
        bb[:, 2:3] * cc[:, 0:1] - bb[:, 0:1] * cc[:, 2:3],
        bb[:, 0:1] * cc[:, 1:2] - bb[:, 1:2] * cc[:, 0:1],
    ], axis=1)
    cb = -0.58273431 * aa + 0.56802827 * bb - 0.54067466 * cc + ca
    x2am = jnp.concatenate([xf[:, 0:12], cb, xf[:, 15:48]], axis=1)
    x2cm_ref[0] = jnp.dot(x2am, p48_ref[...], preferred_element_type=_f32)

    cat = cat_ref[0]
    d2 = None
    for c in range(3):
        row = jnp.broadcast_to(cat[c:c + 1, :], (L, L))
        col = jnp.broadcast_to(ca[:, c:c + 1], (L, L))
        s = (row - col) * (row - col)
        d2 = s if d2 is None else d2 + s
    d_ref[0] = jnp.sqrt(d2 + 1e-6)

    sb = sb_ref[0]
    s_col = sb[:, 0:1].astype(_i32)
    oh = (jax.lax.broadcasted_iota(_i32, (L, 21), 1) == s_col).astype(_f32)
    v_in = jnp.concatenate([oh, sb[:, 1:7], jnp.zeros((L, 5), _f32)], axis=1)
    v = jnp.dot(v_in, wn_ref[...], preferred_element_type=_f32) + bn_ref[0:1, :]
    mu = jnp.mean(v, axis=1, keepdims=True)
    var = jnp.mean((v - mu) ** 2, axis=1, keepdims=True)
    v_ref[0] = (v - mu) / jnp.sqrt(var + 1e-5) * gn_ref[0:1, :] + betan_ref[0:1, :]


_NC, _NS = 2, 16
_NW = _NC * _NS
_ROWS_PER_W = (B * L) // _NW


def _shuf(x, perm):
    dn = jax.lax.GatherDimensionNumbers(
        offset_dims=(), collapsed_slice_dims=(0,), start_index_map=(0,))
    return jax.lax.gather(
        x, perm, dn, (1,),
        mode=jax.lax.GatherScatterMode.PROMISE_IN_BOUNDS)


def _minbcast(x, perms):
    for p in perms:
        x = jnp.minimum(x, _shuf(x, p))
    return x


def _stage2_sc_body(d_hbm, out_hbm, vals_v, vals_v2, idx_v, idx_v2):
    wid = jax.lax.axis_index("s") * _NC + jax.lax.axis_index("c")
    iota = jax.lax.iota(_i32, 16)
    idxs = [iota + t * 16 for t in range(16)]
    perms = [jnp.reshape(iota ^ (1 << s), (16, 1)) for s in range(4)]

    half = _ROWS_PER_W // 2

    def extract(chs, ins, k, r0, r1):
        vs = list(chs)
        ins = list(ins)
        while len(vs) > 1:
            nv, ni = [], []
            for t in range(0, len(vs), 2):
                va, vb, ia, ib = vs[t], vs[t + 1], ins[t], ins[t + 1]
                ta = va <= vb
                nv.append(jnp.where(ta, va, vb))
                ni.append(jnp.where(ta, ia, ib))
            vs, ins = nv, ni
        m = _minbcast(vs[0], perms)
        mi = _minbcast(jnp.where(vs[0] == m, ins[0], jnp.int32(1 << 20)),
                       perms)
        r0 = jnp.where(iota == k, mi, r0)
        r1 = jnp.where(iota == k - 16, mi, r1)
        chs = [jnp.where(idxs[t] == mi, jnp.float32(jnp.inf), chs[t])
               for t in range(16)]
        return chs, r0, r1

    def row_body(j, carry):
        row_a = wid * _ROWS_PER_W + j
        row_b = row_a + half
        pltpu.sync_copy(d_hbm.at[pl.ds(row_a * L, L)], vals_v)
        pltpu.sync_copy(d_hbm.at[pl.ds(row_b * L, L)], vals_v2)
        chunks_a = tuple(vals_v[pl.ds(t * 16, 16)] for t in range(16))
        chunks_b = tuple(vals_v2[pl.ds(t * 16, 16)] for t in range(16))
        z16 = jnp.zeros((16,), _i32)

        def kstep(k, st):
            ca, cb = list(st[0:16]), list(st[16:32])
            a0, a1, b0, b1 = st[32], st[33], st[34], st[35]
            ca, a0, a1 = extract(ca, idxs, k, a0, a1)
            cb, b0, b1 = extract(cb, idxs, k, b0, b1)
            return (*ca, *cb, a0, a1, b0, b1)

        st = jax.lax.fori_loop(0, K, kstep,
                               (*chunks_a, *chunks_b, z16, z16, z16, z16))
        idx_v[pl.ds(0, 16)] = st[32]
        idx_v[pl.ds(16, 16)] = st[33]
        pltpu.sync_copy(idx_v, out_hbm.at[pl.ds(row_a * 32, 32)])
        idx_v2[pl.ds(0, 16)] = st[34]
        idx_v2[pl.ds(16, 16)] = st[35]
        pltpu.sync_copy(idx_v2, out_hbm.at[pl.ds(row_b * 32, 32)])
        return carry

    jax.lax.fori_loop(0, half, row_body, 0)


def _stage3_body(eidx_ref, x2cm_ref, x2blk_ref, wpos_ref, wrbf_ref, expp_ref,
                 expq_ref, rep_ref, mu_ref, be_ref, ge_ref, betae_ref, e_ref):
    ecol = eidx_ref[0, 0][:, 0:1]
    base = pl.program_id(1) * ROWS
    erow = jax.lax.broadcasted_iota(_i32, (NE, 1), 0)
    rr = erow // K
    self_col = base + rr
    iota_l = jax.lax.broadcasted_iota(_i32, (NE, L), 1)
    oh_nb = (iota_l == ecol).astype(_f32)
    oh_self = (jax.lax.broadcasted_iota(_i32, (NE, ROWS), 1) == rr).astype(_f32)
    x2 = x2cm_ref[0]
    xnb = jnp.dot(oh_nb, x2, preferred_element_type=_f32)
    xi = jnp.dot(oh_self, x2blk_ref[0], preferred_element_type=_f32)

    relpos = jnp.clip(ecol - self_col + MAX_REL, 0, 2 * MAX_REL)
    oh_pos = (jax.lax.broadcasted_iota(_i32, (NE, 128), 1) == relpos).astype(_f32)
    acc = jnp.dot(oh_pos, wpos_ref[...], preferred_element_type=_f32) + be_ref[0:1, :]

    d2 = None
    for c in range(3):
        a = jnp.dot(xi[:, c * 16:(c + 1) * 16], expp_ref[...],
                    preferred_element_type=_f32)
        b = jnp.dot(xnb[:, c * 16:(c + 1) * 16], expq_ref[...],
                    preferred_element_type=_f32)
        s = (a - b) * (a - b)
        d2 = s if d2 is None else d2 + s
    dps = jnp.sqrt(d2 + 1e-6) * jnp.float32(0.8)

    mu8 = mu_ref[0:1, :]
    for p in range(NA):
        dpe = jnp.dot(dps[:, p * 16:(p + 1) * 16], rep_ref[...],
                      preferred_element_type=_f32)
        z = dpe - mu8
        rbf = jnp.exp(-(z * z))
        acc = acc + jnp.dot(rbf, wrbf_ref[p], preferred_element_type=_f32)

    m = jnp.mean(acc, axis=1, keepdims=True)
    var = jnp.mean((acc - m) ** 2, axis=1, keepdims=True)
    e_ref[0, 0] = (acc - m) / jnp.sqrt(var + 1e-5) * ge_ref[0:1, :] + betae_ref[0:1, :]


def kernel(X, S, BB_D, mask, W_node, b_node, g_node, beta_node,
           W_edge, b_edge, g_edge, beta_edge):
    del mask
    xf = jnp.pad(X.reshape(B, L, 42), ((0, 0), (0, 0), (0, 6)))
    cat = jnp.pad(jnp.transpose(X[:, :, 1, :], (0, 2, 1)), ((0, 0), (0, 5), (0, 0)))
    sb = jnp.concatenate([S.astype(_f32)[..., None], BB_D.reshape(B, L, 6),
                          jnp.zeros((B, L, 1), _f32)], axis=-1)
    wn = jnp.pad(W_node, ((0, 5), (0, 0)))
    row8 = lambda v: jnp.broadcast_to(v[None, :], (8, v.shape[0]))
    p48 = jnp.asarray(_P48)

    d_mat, x2cm, v_out = pl.pallas_call(
        _stage1_body,
        grid=(B,),
        in_specs=[
            pl.BlockSpec((1, L, 48), lambda b: (b, 0, 0)),
            pl.BlockSpec((1, 8, L), lambda b: (b, 0, 0)),
            pl.BlockSpec((1, L, 8), lambda b: (b, 0, 0)),
            pl.BlockSpec((32, 128), lambda b: (0, 0)),
            pl.BlockSpec((8, 128), lambda b: (0, 0)),
            pl.BlockSpec((8, 128), lambda b: (0, 0)),
            pl.BlockSpec((8, 128), lambda b: (0, 0)),
            pl.BlockSpec((48, 128), lambda b: (0, 0)),
        ],
        out_specs=[
            pl.BlockSpec((1, L, L), lambda b: (b, 0, 0)),
            pl.BlockSpec((1, L, 128), lambda b: (b, 0, 0)),
            pl.BlockSpec((1, L, 128), lambda b: (b, 0, 0)),
        ],
        out_shape=[
            jax.ShapeDtypeStruct((B, L, L), _f32),
            jax.ShapeDtypeStruct((B, L, 128), _f32),
            jax.ShapeDtypeStruct((B, L, 128), _f32),
        ],
    )(xf, cat, sb, wn, row8(b_node), row8(g_node), row8(beta_node), p48)

    sc_mesh = plsc.VectorSubcoreMesh(core_axis_name="c", subcore_axis_name="s")
    eidx_flat = pl.kernel(
        _stage2_sc_body,
        out_type=jax.ShapeDtypeStruct((B * L * 32,), _i32),
        mesh=sc_mesh,
        scratch_types=[
            pltpu.VMEM((L,), _f32),
            pltpu.VMEM((L,), _f32),
            pltpu.VMEM((32,), _i32),
            pltpu.VMEM((32,), _i32),
        ],
    )(d_mat.reshape(B * L * L))

    e_idx = eidx_flat.reshape(B, L, 32)[:, :, :K]
    nblk = L // ROWS
    eidx4 = e_idx.reshape(B, nblk, NE, 1)

    w_pos = jnp.pad(W_edge[:65], ((0, 63), (0, 0)))
    w_rbf = W_edge[65:].reshape(NA, NPP, 128)
    mu_t = row8(jnp.asarray(np.tile(_MU8, NA)))

    e_blocks = pl.pallas_call(
        _stage3_body,
        grid=(B, nblk),
        in_specs=[
            pl.BlockSpec((1, 1, NE, 1), lambda b, i: (b, i, 0, 0)),
            pl.BlockSpec((1, L, 128), lambda b, i: (b, 0, 0)),
            pl.BlockSpec((1, ROWS, 128), lambda b, i: (b, i, 0)),
            pl.BlockSpec((128, 128), lambda b, i: (0, 0)),
            pl.BlockSpec((NA, NPP, 128), lambda b, i: (0, 0, 0)),
            pl.BlockSpec((16, 256), lambda b, i: (0, 0)),
            pl.BlockSpec((16, 256), lambda b, i: (0, 0)),
            pl.BlockSpec((16, NPP), lambda b, i: (0, 0)),
            pl.BlockSpec((8, NPP), lambda b, i: (0, 0)),
            pl.BlockSpec((8, 128), lambda b, i: (0, 0)),
            pl.BlockSpec((8, 128), lambda b, i: (0, 0)),
            pl.BlockSpec((8, 128), lambda b, i: (0, 0)),
        ],
        out_specs=pl.BlockSpec((1, 1, NE, 128), lambda b, i: (b, i, 0, 0)),
        out_shape=jax.ShapeDtypeStruct((B, nblk, NE, 128), _f32),
    )(eidx4, x2cm, x2cm, w_pos, w_rbf, jnp.asarray(_EXP_P), jnp.asarray(_EXP_Q),
      jnp.asarray(_REP16), mu_t, row8(b_edge), row8(g_edge), row8(beta_edge))

    e_out = e_blocks.reshape(B, nblk, ROWS, K, 128).reshape(B, L, K, 128)
    return v_out, e_out, e_idx, X
